# baseline (speedup 1.0000x reference)
import numpy as np
import jax
import jax.numpy as jnp
from jax.experimental import pallas as pl
from jax.experimental.pallas import tpu as pltpu

N = 16384
T = 1024
TB = 128
NB = 4096
THRESHOLD = 0.1
TAU = 1e-8
FIX_CAP = 64

_NN = N // NB
_INV_2PI = np.float32(0.5 / np.pi)
_TWO_OVER_PI = np.float32(2.0 / np.pi)


def _sgn(x):
    b = jax.lax.bitcast_convert_type(x, jnp.uint32)
    s = (b & jnp.uint32(0x80000000)) | jnp.uint32(0x3F800000)
    return jax.lax.bitcast_convert_type(s, jnp.float32)


def _pair_terms(r0, r1, i0, i1):
    p = i0 * i1
    cross = r0 * i1 - i0 * r1
    ab = _sgn(p)
    c = _sgn(cross)
    w = c - ab * c
    g = jnp.minimum(np.float32(TAU) - jnp.abs(cross), -p)
    return w, g


def _vortex_block(real_ref, imag_ref, mean_ref, wind_ref, amb_ref,
                  carry_r, carry_i):
    j = pl.program_id(1)
    r = real_ref[...]
    i = imag_ref[...]

    mag_s = jnp.sum(jnp.sqrt(r * r + i * i), axis=0) * np.float32(1.0 / N)
    w_el, g_el = _pair_terms(r[:-1, :], r[1:, :], i[:-1, :], i[1:, :])
    ws = jnp.sum(w_el, axis=0)
    g = jnp.max(g_el, axis=0)

    @pl.when(j == 0)
    def _init():
        theta_first = jnp.arctan2(i[0, :], r[0, :])
        mean_ref[...] = mag_s[None, :]
        wind_ref[...] = (ws + _sgn(i[0, :]) - theta_first * _TWO_OVER_PI)[None, :]
        amb_ref[...] = g[None, :]

    @pl.when(j > 0)
    def _accum():
        wb, gb = _pair_terms(carry_r[0, :], r[0, :], carry_i[0, :], i[0, :])
        mean_ref[...] += mag_s[None, :]
        wind_ref[...] += (ws + wb)[None, :]
        amb_ref[...] = jnp.maximum(amb_ref[...], jnp.maximum(g, gb)[None, :])

    @pl.when(j == _NN - 1)
    def _finish():
        theta_last = jnp.arctan2(i[-1, :], r[-1, :])
        acc = wind_ref[0, :] - _sgn(i[-1, :]) + theta_last * _TWO_OVER_PI
        wind_ref[...] = (acc * np.float32(0.25))[None, :]

    carry_r[0, :] = r[-1, :]
    carry_i[0, :] = i[-1, :]


@jax.jit
def kernel(field_real, field_imag):
    mean_mag, winding, g = pl.pallas_call(
        _vortex_block,
        grid=(T // TB, _NN),
        in_specs=[
            pl.BlockSpec((NB, TB), lambda t, j: (j, t)),
            pl.BlockSpec((NB, TB), lambda t, j: (j, t)),
        ],
        out_specs=[
            pl.BlockSpec((1, TB), lambda t, j: (0, t)),
            pl.BlockSpec((1, TB), lambda t, j: (0, t)),
            pl.BlockSpec((1, TB), lambda t, j: (0, t)),
        ],
        out_shape=[
            jax.ShapeDtypeStruct((1, T), jnp.float32),
            jax.ShapeDtypeStruct((1, T), jnp.float32),
            jax.ShapeDtypeStruct((1, T), jnp.float32),
        ],
        scratch_shapes=[
            pltpu.VMEM((8, TB), jnp.float32),
            pltpu.VMEM((8, TB), jnp.float32),
        ],
    )(field_real, field_imag)
    mean_mag = mean_mag.reshape(T)
    winding = winding.reshape(T)
    g = g.reshape(T)

    flag = (g > 0.0) | (jnp.abs(jnp.abs(winding) - 0.5) < 0.01)
    (idx,) = jnp.nonzero(flag, size=FIX_CAP, fill_value=0)
    phases = jnp.arctan2(field_imag[:, idx], field_real[:, idx])
    pd = jnp.diff(phases, axis=0)
    pd = jnp.where(pd > np.pi, pd - 2.0 * np.pi, pd)
    pd = jnp.where(pd < -np.pi, pd + 2.0 * np.pi, pd)
    w_fix = jnp.sum(pd, axis=0) / (2.0 * np.pi)
    winding = winding.at[idx].set(w_fix)

    is_v = (mean_mag < THRESHOLD) & (jnp.abs(winding) > 0.5)
    return (is_v.astype(jnp.int32), jnp.where(is_v, winding, 0.0))


# speedup vs baseline: 1.6941x; 1.6941x over previous
import numpy as np
import jax
import jax.numpy as jnp
from jax.experimental import pallas as pl
from jax.experimental.pallas import tpu as pltpu

N = 16384
T = 1024
TB = 128
NB = 4096
THRESHOLD = 0.1
TAU = 1e-8
FIX_CAP = 64

_NN = N // NB
_TWO_OVER_PI = np.float32(2.0 / np.pi)


def _sgn(x):
    b = jax.lax.bitcast_convert_type(x, jnp.uint32)
    s = (b & jnp.uint32(0x80000000)) | jnp.uint32(0x3F800000)
    return jax.lax.bitcast_convert_type(s, jnp.float32)


def _pair_terms(r0, r1, i0, i1):
    p = i0 * i1
    cross = r0 * i1 - i0 * r1
    ab = _sgn(p)
    c = _sgn(cross)
    w = c - ab * c
    amb = jnp.minimum(np.float32(TAU) - jnp.abs(cross), -p) > 0.0
    return w, amb


def _vortex_block(real_ref, imag_ref, mean_ref, wind_ref, fix_ref,
                  carry_r, carry_i):
    j = pl.program_id(1)
    r = real_ref[...]
    i = imag_ref[...]

    mag_s = jnp.sum(jnp.sqrt(r * r + i * i), axis=0) * np.float32(1.0 / N)
    w_el, amb = _pair_terms(r[:-1, :], r[1:, :], i[:-1, :], i[1:, :])
    ws = jnp.sum(w_el, axis=0)
    rows = jax.lax.broadcasted_iota(jnp.int32, w_el.shape, 0)
    fi = jnp.max(jnp.where(amb, rows, -1), axis=0)
    base = j * NB
    fi = jnp.where(fi >= 0, fi + base, -1)

    @pl.when(j == 0)
    def _init():
        theta_first = jnp.arctan2(i[0, :], r[0, :])
        mean_ref[...] = mag_s[None, :]
        wind_ref[...] = (ws + _sgn(i[0, :]) - theta_first * _TWO_OVER_PI)[None, :]
        fix_ref[...] = fi[None, :]

    @pl.when(j > 0)
    def _accum():
        wb, amb_b = _pair_terms(carry_r[0, :], r[0, :], carry_i[0, :], i[0, :])
        fb = jnp.where(amb_b, base - 1, -1)
        mean_ref[...] += mag_s[None, :]
        wind_ref[...] += (ws + wb)[None, :]
        fix_ref[...] = jnp.maximum(fix_ref[...], jnp.maximum(fi, fb)[None, :])

    @pl.when(j == _NN - 1)
    def _finish():
        theta_last = jnp.arctan2(i[-1, :], r[-1, :])
        acc = wind_ref[0, :] - _sgn(i[-1, :]) + theta_last * _TWO_OVER_PI
        wind_ref[...] = (acc * np.float32(0.25))[None, :]

    carry_r[0, :] = r[-1, :]
    carry_i[0, :] = i[-1, :]


@jax.jit
def kernel(field_real, field_imag):
    mean_mag, winding, fix = pl.pallas_call(
        _vortex_block,
        grid=(T // TB, _NN),
        in_specs=[
            pl.BlockSpec((NB, TB), lambda t, j: (j, t)),
            pl.BlockSpec((NB, TB), lambda t, j: (j, t)),
        ],
        out_specs=[
            pl.BlockSpec((1, TB), lambda t, j: (0, t)),
            pl.BlockSpec((1, TB), lambda t, j: (0, t)),
            pl.BlockSpec((1, TB), lambda t, j: (0, t)),
        ],
        out_shape=[
            jax.ShapeDtypeStruct((1, T), jnp.float32),
            jax.ShapeDtypeStruct((1, T), jnp.float32),
            jax.ShapeDtypeStruct((1, T), jnp.int32),
        ],
        scratch_shapes=[
            pltpu.VMEM((8, TB), jnp.float32),
            pltpu.VMEM((8, TB), jnp.float32),
        ],
    )(field_real, field_imag)
    mean_mag = mean_mag.reshape(T)
    winding = winding.reshape(T)
    fix = fix.reshape(T)

    fv, cols = jax.lax.top_k(fix, FIX_CAP)
    n = jnp.clip(fv, 0, N - 2)
    r0 = field_real[n, cols]
    r1 = field_real[n + 1, cols]
    i0 = field_imag[n, cols]
    i1 = field_imag[n + 1, cols]
    pd = jnp.arctan2(i1, r1) - jnp.arctan2(i0, r0)
    ref_cnt = jnp.where(pd < -np.pi, 1.0, 0.0) - jnp.where(pd > np.pi, 1.0, 0.0)
    a = _sgn(i0)
    b = _sgn(i1)
    c = _sgn(r0 * i1 - i0 * r1)
    our_cnt = (a - b + c - a * b * c) * np.float32(0.25)
    winding = winding.at[cols].set(winding[cols] + ref_cnt - our_cnt)

    is_v = (mean_mag < THRESHOLD) & (jnp.abs(winding) > 0.5)
    return (is_v.astype(jnp.int32), jnp.where(is_v, winding, 0.0))
